# baseline (speedup 1.0000x reference)
import functools

import jax
import jax.numpy as jnp
from jax import lax
from jax.experimental import pallas as pl
from jax.experimental.pallas import tpu as pltpu
from jax.experimental.pallas import tpu_sc as plsc

N = 10000
E = 320000
D = 128
H = 128
L = 4
G = 64
OUT = 12
BN_EPS = 1e-5

NC = 2
NS = 16
NW = NC * NS
EW = E // NW
C = 128
K = 80
EWP = K * C
GRP = 8
NG = K // GRP
NP = 10240
RPT = NP // NS

@functools.lru_cache(maxsize=1)
def _get_sc_segment_sum():
    mesh = plsc.VectorSubcoreMesh(core_axis_name="c", subcore_axis_name="s",
                                  num_cores=NC, num_subcores=NS)

    @functools.partial(
        pl.kernel,
        out_type=jax.ShapeDtypeStruct((NC, NP, H), jnp.float32),
        mesh=mesh,
        scratch_types=[
            pltpu.VMEM((K, C), jnp.int32),
            pltpu.VMEM((GRP, C), jnp.int32),
            pltpu.VMEM((C, H), jnp.float32),
            pltpu.VMEM_SHARED((NP, H), jnp.float32),
            pltpu.SemaphoreType.DMA,
            pltpu.SemaphoreType.DMA,
        ],
    )
    def sc_segment_sum(h_hbm, src_hbm, dst_hbm, zeros_hbm, out_hbm,
                       src_v, didx, rows, agg_sh, sem_g, sem_z):
        c = lax.axis_index("c")
        s = lax.axis_index("s")
        w = c * NS + s
        pltpu.async_copy(zeros_hbm.at[pl.ds(s * RPT, RPT)],
                         agg_sh.at[pl.ds(s * RPT, RPT)], sem_z)
        pltpu.async_copy(src_hbm.at[w], src_v, sem_g)
        pltpu.make_async_copy(src_hbm.at[w], src_v, sem_g).wait()
        pltpu.make_async_copy(zeros_hbm.at[pl.ds(s * RPT, RPT)],
                              agg_sh.at[pl.ds(s * RPT, RPT)], sem_z).wait()
        plsc.subcore_barrier()

        def step(g, carry):
            pltpu.sync_copy(dst_hbm.at[w, g], didx)
            for j in range(GRP):
                k = g * GRP + j
                pltpu.async_copy(h_hbm.at[src_v.at[k]], rows, sem_g)
                pltpu.make_async_copy(h_hbm.at[src_v.at[0]], rows,
                                      sem_g).wait()
                pltpu.sync_copy(rows, agg_sh.at[didx.at[j]], add=True)
            return carry

        lax.fori_loop(0, NG, step, 0)
        plsc.subcore_barrier()
        pltpu.sync_copy(agg_sh.at[pl.ds(s * RPT, RPT)],
                        out_hbm.at[c, pl.ds(s * RPT, RPT)])

    return sc_segment_sum


def _embed_body(x_ref, w_ref, b_ref, out_ref):
    out_ref[...] = (
        jnp.dot(x_ref[...], w_ref[...], preferred_element_type=jnp.float32)
        + b_ref[...]
    )


def _layer_body(h_ref, agg_ref, w1_ref, b1_ref, w2_ref, b2_ref,
                g_ref, be_ref, out_ref):
    hsum = h_ref[...] + agg_ref[0, :N] + agg_ref[1, :N]
    h2 = jnp.dot(hsum, w1_ref[...], preferred_element_type=jnp.float32)
    h2 = jnp.maximum(h2 + b1_ref[...], 0.0)
    h3 = jnp.dot(h2, w2_ref[...], preferred_element_type=jnp.float32)
    h3 = h3 + b2_ref[...]
    mean = jnp.mean(h3, axis=0, keepdims=True)
    var = jnp.mean(jnp.square(h3 - mean), axis=0, keepdims=True)
    h3 = g_ref[...] * (h3 - mean) * lax.rsqrt(var + BN_EPS) + be_ref[...]
    out_ref[...] = jnp.maximum(h3, 0.0)


def _last_layer_body(h_ref, agg_ref, w1_ref, b1_ref, w2_ref, b2_ref,
                     g_ref, be_ref, batch_ref, hw1_ref, hb1_ref, hw2_ref,
                     hb2_ref, out_ref):
    hsum = h_ref[...] + agg_ref[0, :N] + agg_ref[1, :N]
    h2 = jnp.dot(hsum, w1_ref[...], preferred_element_type=jnp.float32)
    h2 = jnp.maximum(h2 + b1_ref[...], 0.0)
    h3 = jnp.dot(h2, w2_ref[...], preferred_element_type=jnp.float32)
    h3 = h3 + b2_ref[...]
    mean = jnp.mean(h3, axis=0, keepdims=True)
    var = jnp.mean(jnp.square(h3 - mean), axis=0, keepdims=True)
    h3 = g_ref[...] * (h3 - mean) * lax.rsqrt(var + BN_EPS) + be_ref[...]
    h4 = jnp.maximum(h3, 0.0)
    ids = lax.broadcasted_iota(jnp.int32, (G, N), 0)
    onehot = (batch_ref[...] == ids).astype(jnp.float32)
    g = jnp.dot(onehot, h4, preferred_element_type=jnp.float32)
    g = jnp.dot(g, hw1_ref[...], preferred_element_type=jnp.float32)
    g = jnp.maximum(g + hb1_ref[...], 0.0)
    g = jnp.dot(g, hw2_ref[...], preferred_element_type=jnp.float32)
    out_ref[...] = g + hb2_ref[...]


def _pad_edges(edge_index):
    pad = EWP - EW
    src_w = edge_index[0].reshape(NW, EW)
    dst_w = edge_index[1].reshape(NW, EW)
    pad_src = (jnp.arange(NW * pad, dtype=jnp.int32) * 37 % N).reshape(NW, pad)
    pad_dst = jnp.broadcast_to(
        N + jnp.arange(pad, dtype=jnp.int32), (NW, pad))
    src = jnp.concatenate([src_w, pad_src], axis=1).reshape(NW, K, C)
    dst = jnp.concatenate([dst_w, pad_dst], axis=1).reshape(NW, NG, GRP, C)
    return src, dst


def kernel(x, edge_index, batch, node_w, node_b, conv_w1, conv_b1, conv_w2,
           conv_b2, bn_gamma, bn_beta, head_w1, head_b1, head_w2, head_b2):
    src, dst = _pad_edges(edge_index)
    zeros = jnp.zeros((NP, H), dtype=jnp.float32)
    batch2 = batch[None, :]

    h = pl.pallas_call(
        _embed_body,
        out_shape=jax.ShapeDtypeStruct((N, H), jnp.float32),
    )(x, node_w, node_b[None, :])

    for i in range(L - 1):
        agg = _get_sc_segment_sum()(h, src, dst, zeros)
        h = pl.pallas_call(
            _layer_body,
            out_shape=jax.ShapeDtypeStruct((N, H), jnp.float32),
        )(h, agg, conv_w1[i], conv_b1[i][None, :], conv_w2[i],
          conv_b2[i][None, :], bn_gamma[i][None, :], bn_beta[i][None, :])

    agg = _get_sc_segment_sum()(h, src, dst, zeros)
    w2p = jnp.pad(head_w2, ((0, 0), (0, H - OUT)))
    b2p = jnp.pad(head_b2, (0, H - OUT))
    i = L - 1
    out = pl.pallas_call(
        _last_layer_body,
        out_shape=jax.ShapeDtypeStruct((G, H), jnp.float32),
    )(h, agg, conv_w1[i], conv_b1[i][None, :], conv_w2[i],
      conv_b2[i][None, :], bn_gamma[i][None, :], bn_beta[i][None, :],
      batch2, head_w1, head_b1[None, :], w2p, b2p[None, :])

    return out[:, :OUT]


# speedup vs baseline: 1.3800x; 1.3800x over previous
import functools

import jax
import jax.numpy as jnp
from jax import lax
from jax.experimental import pallas as pl
from jax.experimental.pallas import tpu as pltpu
from jax.experimental.pallas import tpu_sc as plsc

N = 10000
E = 320000
D = 128
H = 128
L = 4
G = 64
OUT = 12
BN_EPS = 1e-5

NC = 2
NS = 16
NW = NC * NS
EW = E // NW
C = 128
K = 80
EWP = K * C
GRP = 8
NG = K // GRP
NP = 10240
RPT = NP // NS

@functools.lru_cache(maxsize=1)
def _get_sc_segment_sum():
    mesh = plsc.VectorSubcoreMesh(core_axis_name="c", subcore_axis_name="s",
                                  num_cores=NC, num_subcores=NS)

    @functools.partial(
        pl.kernel,
        out_type=jax.ShapeDtypeStruct((NC, NP, H), jnp.float32),
        mesh=mesh,
        scratch_types=[
            pltpu.VMEM((K, C), jnp.int32),
            pltpu.VMEM((GRP, C), jnp.int32),
            pltpu.VMEM((C, H), jnp.float32),
            pltpu.VMEM((C, H), jnp.float32),
            pltpu.VMEM_SHARED((NP, H), jnp.float32),
            pltpu.SemaphoreType.DMA,
            pltpu.SemaphoreType.DMA,
            pltpu.SemaphoreType.DMA,
        ],
    )
    def sc_segment_sum(h_hbm, src_hbm, dst_hbm, zeros_hbm, out_hbm,
                       src_v, didx, rows_a, rows_b, agg_sh,
                       sem_ga, sem_gb, sem_z):
        c = lax.axis_index("c")
        s = lax.axis_index("s")
        w = c * NS + s
        pltpu.async_copy(zeros_hbm.at[pl.ds(s * RPT, RPT)],
                         agg_sh.at[pl.ds(s * RPT, RPT)], sem_z)
        pltpu.async_copy(src_hbm.at[w], src_v, sem_ga)
        pltpu.make_async_copy(src_hbm.at[w], src_v, sem_ga).wait()
        pltpu.make_async_copy(zeros_hbm.at[pl.ds(s * RPT, RPT)],
                              agg_sh.at[pl.ds(s * RPT, RPT)], sem_z).wait()
        plsc.subcore_barrier()

        rows = (rows_a, rows_b)
        gsems = (sem_ga, sem_gb)

        def step(g, carry):
            pltpu.sync_copy(dst_hbm.at[w, g], didx)
            pltpu.async_copy(h_hbm.at[src_v.at[g * GRP]], rows_a, sem_ga)
            for j in range(GRP):
                k = g * GRP + j
                if j + 1 < GRP:
                    pltpu.async_copy(h_hbm.at[src_v.at[k + 1]],
                                     rows[(j + 1) % 2], gsems[(j + 1) % 2])
                pltpu.make_async_copy(h_hbm.at[src_v.at[0]],
                                      rows[j % 2], gsems[j % 2]).wait()
                pltpu.sync_copy(rows[j % 2], agg_sh.at[didx.at[j]],
                                add=True)
            return carry

        lax.fori_loop(0, NG, step, 0)
        plsc.subcore_barrier()
        pltpu.sync_copy(agg_sh.at[pl.ds(s * RPT, RPT)],
                        out_hbm.at[c, pl.ds(s * RPT, RPT)])

    return sc_segment_sum


def _embed_body(x_ref, w_ref, b_ref, out_ref):
    out_ref[...] = (
        jnp.dot(x_ref[...], w_ref[...], preferred_element_type=jnp.float32)
        + b_ref[...]
    )


def _layer_body(h_ref, agg_ref, w1_ref, b1_ref, w2_ref, b2_ref,
                g_ref, be_ref, out_ref):
    hsum = h_ref[...] + agg_ref[0, :N] + agg_ref[1, :N]
    h2 = jnp.dot(hsum, w1_ref[...], preferred_element_type=jnp.float32)
    h2 = jnp.maximum(h2 + b1_ref[...], 0.0)
    h3 = jnp.dot(h2, w2_ref[...], preferred_element_type=jnp.float32)
    h3 = h3 + b2_ref[...]
    mean = jnp.mean(h3, axis=0, keepdims=True)
    var = jnp.mean(jnp.square(h3 - mean), axis=0, keepdims=True)
    h3 = g_ref[...] * (h3 - mean) * lax.rsqrt(var + BN_EPS) + be_ref[...]
    out_ref[...] = jnp.maximum(h3, 0.0)


def _last_layer_body(h_ref, agg_ref, w1_ref, b1_ref, w2_ref, b2_ref,
                     g_ref, be_ref, batch_ref, hw1_ref, hb1_ref, hw2_ref,
                     hb2_ref, out_ref):
    hsum = h_ref[...] + agg_ref[0, :N] + agg_ref[1, :N]
    h2 = jnp.dot(hsum, w1_ref[...], preferred_element_type=jnp.float32)
    h2 = jnp.maximum(h2 + b1_ref[...], 0.0)
    h3 = jnp.dot(h2, w2_ref[...], preferred_element_type=jnp.float32)
    h3 = h3 + b2_ref[...]
    mean = jnp.mean(h3, axis=0, keepdims=True)
    var = jnp.mean(jnp.square(h3 - mean), axis=0, keepdims=True)
    h3 = g_ref[...] * (h3 - mean) * lax.rsqrt(var + BN_EPS) + be_ref[...]
    h4 = jnp.maximum(h3, 0.0)
    ids = lax.broadcasted_iota(jnp.int32, (G, N), 0)
    onehot = (batch_ref[...] == ids).astype(jnp.float32)
    g = jnp.dot(onehot, h4, preferred_element_type=jnp.float32)
    g = jnp.dot(g, hw1_ref[...], preferred_element_type=jnp.float32)
    g = jnp.maximum(g + hb1_ref[...], 0.0)
    g = jnp.dot(g, hw2_ref[...], preferred_element_type=jnp.float32)
    out_ref[...] = g + hb2_ref[...]


def _pad_edges(edge_index):
    pad = EWP - EW
    src_w = edge_index[0].reshape(NW, EW)
    dst_w = edge_index[1].reshape(NW, EW)
    pad_src = (jnp.arange(NW * pad, dtype=jnp.int32) * 37 % N).reshape(NW, pad)
    pad_dst = jnp.broadcast_to(
        N + jnp.arange(pad, dtype=jnp.int32), (NW, pad))
    src = jnp.concatenate([src_w, pad_src], axis=1).reshape(NW, K, C)
    dst = jnp.concatenate([dst_w, pad_dst], axis=1).reshape(NW, NG, GRP, C)
    return src, dst


def kernel(x, edge_index, batch, node_w, node_b, conv_w1, conv_b1, conv_w2,
           conv_b2, bn_gamma, bn_beta, head_w1, head_b1, head_w2, head_b2):
    src, dst = _pad_edges(edge_index)
    zeros = jnp.zeros((NP, H), dtype=jnp.float32)
    batch2 = batch[None, :]

    h = pl.pallas_call(
        _embed_body,
        out_shape=jax.ShapeDtypeStruct((N, H), jnp.float32),
    )(x, node_w, node_b[None, :])

    for i in range(L - 1):
        agg = _get_sc_segment_sum()(h, src, dst, zeros)
        h = pl.pallas_call(
            _layer_body,
            out_shape=jax.ShapeDtypeStruct((N, H), jnp.float32),
        )(h, agg, conv_w1[i], conv_b1[i][None, :], conv_w2[i],
          conv_b2[i][None, :], bn_gamma[i][None, :], bn_beta[i][None, :])

    agg = _get_sc_segment_sum()(h, src, dst, zeros)
    w2p = jnp.pad(head_w2, ((0, 0), (0, H - OUT)))
    b2p = jnp.pad(head_b2, (0, H - OUT))
    i = L - 1
    out = pl.pallas_call(
        _last_layer_body,
        out_shape=jax.ShapeDtypeStruct((G, H), jnp.float32),
    )(h, agg, conv_w1[i], conv_b1[i][None, :], conv_w2[i],
      conv_b2[i][None, :], bn_gamma[i][None, :], bn_beta[i][None, :],
      batch2, head_w1, head_b1[None, :], w2p, b2p[None, :])

    return out[:, :OUT]


# speedup vs baseline: 1.5779x; 1.1434x over previous
import functools

import jax
import jax.numpy as jnp
from jax import lax
from jax.experimental import pallas as pl
from jax.experimental.pallas import tpu as pltpu
from jax.experimental.pallas import tpu_sc as plsc

N = 10000
E = 320000
D = 128
H = 128
L = 4
G = 64
OUT = 12
BN_EPS = 1e-5

NC = 2
NS = 16
NW = NC * NS
EW = E // NW
C = 80
K = 125
GRP = 25
NG = K // GRP
NP = 10112
RPT = NP // NS

@functools.lru_cache(maxsize=1)
def _get_sc_segment_sum():
    mesh = plsc.VectorSubcoreMesh(core_axis_name="c", subcore_axis_name="s",
                                  num_cores=NC, num_subcores=NS)

    @functools.partial(
        pl.kernel,
        out_type=jax.ShapeDtypeStruct((NC, NP, H), jnp.float32),
        mesh=mesh,
        scratch_types=[
            pltpu.VMEM((GRP, C), jnp.int32),
            pltpu.VMEM((GRP, C), jnp.int32),
            pltpu.VMEM((C, H), jnp.float32),
            pltpu.VMEM((C, H), jnp.float32),
            pltpu.VMEM((C, H), jnp.float32),
            pltpu.VMEM_SHARED((NP, H), jnp.float32),
            pltpu.SemaphoreType.DMA,
            pltpu.SemaphoreType.DMA,
            pltpu.SemaphoreType.DMA,
            pltpu.SemaphoreType.DMA,
            pltpu.SemaphoreType.DMA,
            pltpu.SemaphoreType.DMA,
            pltpu.SemaphoreType.DMA,
        ],
    )
    def sc_segment_sum(h_hbm, src_hbm, dst_hbm, zeros_hbm, out_hbm,
                       srcg, didx, rows_a, rows_b, rows_c, agg_sh,
                       sem_ga, sem_gb, sem_gc, sem_sa, sem_sb, sem_sc,
                       sem_z):
        c = lax.axis_index("c")
        s = lax.axis_index("s")
        w = c * NS + s
        pltpu.async_copy(zeros_hbm.at[pl.ds(s * RPT, RPT)],
                         agg_sh.at[pl.ds(s * RPT, RPT)], sem_z)
        pltpu.make_async_copy(zeros_hbm.at[pl.ds(s * RPT, RPT)],
                              agg_sh.at[pl.ds(s * RPT, RPT)], sem_z).wait()
        plsc.subcore_barrier()

        rows = (rows_a, rows_b, rows_c)
        gsems = (sem_ga, sem_gb, sem_gc)
        ssems = (sem_sa, sem_sb, sem_sc)

        def step(g, carry):
            pltpu.sync_copy(src_hbm.at[w, g], srcg)
            pltpu.async_copy(h_hbm.at[srcg.at[0]], rows_a, sem_ga)
            pltpu.sync_copy(dst_hbm.at[w, g], didx)
            for j in range(GRP):
                if j + 1 < GRP:
                    nb = (j + 1) % 3
                    if j >= 2:
                        pltpu.make_async_copy(
                            rows[nb], agg_sh.at[didx.at[j - 2]],
                            ssems[nb]).wait()
                    pltpu.async_copy(h_hbm.at[srcg.at[j + 1]],
                                     rows[nb], gsems[nb])
                pltpu.make_async_copy(h_hbm.at[srcg.at[0]],
                                      rows[j % 3], gsems[j % 3]).wait()
                pltpu.async_copy(rows[j % 3], agg_sh.at[didx.at[j]],
                                 ssems[j % 3], add=True)
            for j in range(GRP - 3, GRP):
                pltpu.make_async_copy(rows[j % 3], agg_sh.at[didx.at[0]],
                                      ssems[j % 3]).wait()
            return carry

        lax.fori_loop(0, NG, step, 0)
        plsc.subcore_barrier()
        pltpu.sync_copy(agg_sh.at[pl.ds(s * RPT, RPT)],
                        out_hbm.at[c, pl.ds(s * RPT, RPT)])

    return sc_segment_sum


def _embed_body(x_ref, w_ref, b_ref, out_ref):
    out_ref[...] = (
        jnp.dot(x_ref[...], w_ref[...], preferred_element_type=jnp.float32)
        + b_ref[...]
    )


def _layer_body(h_ref, agg_ref, w1_ref, b1_ref, w2_ref, b2_ref,
                g_ref, be_ref, out_ref):
    hsum = h_ref[...] + agg_ref[0, :N] + agg_ref[1, :N]
    h2 = jnp.dot(hsum, w1_ref[...], preferred_element_type=jnp.float32)
    h2 = jnp.maximum(h2 + b1_ref[...], 0.0)
    h3 = jnp.dot(h2, w2_ref[...], preferred_element_type=jnp.float32)
    h3 = h3 + b2_ref[...]
    mean = jnp.mean(h3, axis=0, keepdims=True)
    var = jnp.mean(jnp.square(h3 - mean), axis=0, keepdims=True)
    h3 = g_ref[...] * (h3 - mean) * lax.rsqrt(var + BN_EPS) + be_ref[...]
    out_ref[...] = jnp.maximum(h3, 0.0)


def _last_layer_body(h_ref, agg_ref, w1_ref, b1_ref, w2_ref, b2_ref,
                     g_ref, be_ref, batch_ref, hw1_ref, hb1_ref, hw2_ref,
                     hb2_ref, out_ref):
    hsum = h_ref[...] + agg_ref[0, :N] + agg_ref[1, :N]
    h2 = jnp.dot(hsum, w1_ref[...], preferred_element_type=jnp.float32)
    h2 = jnp.maximum(h2 + b1_ref[...], 0.0)
    h3 = jnp.dot(h2, w2_ref[...], preferred_element_type=jnp.float32)
    h3 = h3 + b2_ref[...]
    mean = jnp.mean(h3, axis=0, keepdims=True)
    var = jnp.mean(jnp.square(h3 - mean), axis=0, keepdims=True)
    h3 = g_ref[...] * (h3 - mean) * lax.rsqrt(var + BN_EPS) + be_ref[...]
    h4 = jnp.maximum(h3, 0.0)
    ids = lax.broadcasted_iota(jnp.int32, (G, N), 0)
    onehot = (batch_ref[...] == ids).astype(jnp.float32)
    g = jnp.dot(onehot, h4, preferred_element_type=jnp.float32)
    g = jnp.dot(g, hw1_ref[...], preferred_element_type=jnp.float32)
    g = jnp.maximum(g + hb1_ref[...], 0.0)
    g = jnp.dot(g, hw2_ref[...], preferred_element_type=jnp.float32)
    out_ref[...] = g + hb2_ref[...]


def _chunk_edges(edge_index):
    src = edge_index[0].reshape(NW, NG, GRP, C)
    dst = edge_index[1].reshape(NW, NG, GRP, C)
    return src, dst


def kernel(x, edge_index, batch, node_w, node_b, conv_w1, conv_b1, conv_w2,
           conv_b2, bn_gamma, bn_beta, head_w1, head_b1, head_w2, head_b2):
    src, dst = _chunk_edges(edge_index)
    zeros = jnp.zeros((NP, H), dtype=jnp.float32)
    batch2 = batch[None, :]

    h = pl.pallas_call(
        _embed_body,
        out_shape=jax.ShapeDtypeStruct((N, H), jnp.float32),
    )(x, node_w, node_b[None, :])

    for i in range(L - 1):
        agg = _get_sc_segment_sum()(h, src, dst, zeros)
        h = pl.pallas_call(
            _layer_body,
            out_shape=jax.ShapeDtypeStruct((N, H), jnp.float32),
        )(h, agg, conv_w1[i], conv_b1[i][None, :], conv_w2[i],
          conv_b2[i][None, :], bn_gamma[i][None, :], bn_beta[i][None, :])

    agg = _get_sc_segment_sum()(h, src, dst, zeros)
    w2p = jnp.pad(head_w2, ((0, 0), (0, H - OUT)))
    b2p = jnp.pad(head_b2, (0, H - OUT))
    i = L - 1
    out = pl.pallas_call(
        _last_layer_body,
        out_shape=jax.ShapeDtypeStruct((G, H), jnp.float32),
    )(h, agg, conv_w1[i], conv_b1[i][None, :], conv_w2[i],
      conv_b2[i][None, :], bn_gamma[i][None, :], bn_beta[i][None, :],
      batch2, head_w1, head_b1[None, :], w2p, b2p[None, :])

    return out[:, :OUT]
